# Initial kernel scaffold; baseline (speedup 1.0000x reference)
import jax
import jax.numpy as jnp
from jax.experimental import pallas as pl


def kernel(x, table):
    raise NotImplementedError("write your pallas kernel here")


# speedup vs baseline: 1.6845x; 1.6845x over previous
import functools

import jax
import jax.numpy as jnp
from jax import lax
from jax.experimental import pallas as pl
from jax.experimental.pallas import tpu as pltpu
from jax.experimental.pallas import tpu_sc as plsc

DIM = 64
IDXW = 128


@functools.cache
def _make_kernel(n_rows: int):
    info = plsc.get_sparse_core_info()
    nc, ns = info.num_cores, info.num_subcores
    nw = nc * ns
    assert n_rows % nw == 0
    rows_per_w = n_rows // nw
    mesh = plsc.VectorSubcoreMesh(core_axis_name="c", subcore_axis_name="s")

    @functools.partial(
        pl.kernel,
        mesh=mesh,
        out_type=jax.ShapeDtypeStruct((n_rows * IDXW, DIM), jnp.float32),
        scratch_types=[
            pltpu.VMEM((rows_per_w, IDXW), jnp.int32),
            pltpu.VMEM((IDXW, DIM), jnp.float32),
            pltpu.SemaphoreType.DMA,
        ],
        compiler_params=pltpu.CompilerParams(use_tc_tiling_on_sc=False),
    )
    def k(idx_hbm, table_hbm, out_hbm, idx_v, rows_v, sem):
        wid = lax.axis_index("s") * nc + lax.axis_index("c")
        row0 = wid * rows_per_w
        pltpu.sync_copy(idx_hbm.at[pl.ds(row0, rows_per_w)], idx_v)

        def body(j, carry):
            pltpu.async_copy(table_hbm.at[idx_v.at[j]], rows_v, sem).wait()
            pltpu.sync_copy(rows_v, out_hbm.at[pl.ds((row0 + j) * IDXW, IDXW)])
            return carry

        lax.fori_loop(0, rows_per_w, body, 0)

    return k


def kernel(x, table):
    b, l = x.shape
    flat = x.reshape(-1)
    n_rows = flat.shape[0] // IDXW
    idx2d = flat.reshape(n_rows, IDXW)
    out = _make_kernel(n_rows)(idx2d, table)
    return out.reshape(b, l, DIM)


# speedup vs baseline: 1.8732x; 1.1120x over previous
import functools

import jax
import jax.numpy as jnp
from jax import lax
from jax.experimental import pallas as pl
from jax.experimental.pallas import tpu as pltpu
from jax.experimental.pallas import tpu_sc as plsc

DIM = 64
IDXW = 128
NBUF = 8


@functools.cache
def _make_kernel(n_rows: int):
    info = plsc.get_sparse_core_info()
    nc, ns = info.num_cores, info.num_subcores
    nw = nc * ns
    assert n_rows % nw == 0
    rows_per_w = n_rows // nw
    assert rows_per_w % NBUF == 0 and rows_per_w > NBUF
    mesh = plsc.VectorSubcoreMesh(core_axis_name="c", subcore_axis_name="s")

    @functools.partial(
        pl.kernel,
        mesh=mesh,
        out_type=jax.ShapeDtypeStruct((n_rows * IDXW, DIM), jnp.float32),
        scratch_types=[
            pltpu.VMEM((rows_per_w, IDXW), jnp.int32),
            pltpu.VMEM((NBUF, IDXW, DIM), jnp.float32),
            pltpu.SemaphoreType.DMA((NBUF,)),
            pltpu.SemaphoreType.DMA((NBUF,)),
        ],
        compiler_params=pltpu.CompilerParams(use_tc_tiling_on_sc=False),
    )
    def k(idx_hbm, table_hbm, out_hbm, idx_v, rows_v, sem_g, sem_s):
        wid = lax.axis_index("s") * nc + lax.axis_index("c")
        row0 = wid * rows_per_w
        pltpu.sync_copy(idx_hbm.at[pl.ds(row0, rows_per_w)], idx_v)

        def gather_start(j, b):
            pltpu.async_copy(table_hbm.at[idx_v.at[j]], rows_v.at[b], sem_g.at[b])

        def gather_wait(j, b):
            pltpu.make_async_copy(
                table_hbm.at[idx_v.at[j]], rows_v.at[b], sem_g.at[b]
            ).wait()

        def store_start(j, b):
            pltpu.async_copy(
                rows_v.at[b], out_hbm.at[pl.ds((row0 + j) * IDXW, IDXW)], sem_s.at[b]
            )

        def store_wait(j, b):
            pltpu.make_async_copy(
                rows_v.at[b], out_hbm.at[pl.ds((row0 + j) * IDXW, IDXW)], sem_s.at[b]
            ).wait()

        for b in range(NBUF):
            gather_start(b, b)

        @pl.loop(0, rows_per_w - NBUF, step=NBUF)
        def _(g):
            for b in range(NBUF):
                gather_wait(g + b, b)
                store_start(g + b, b)
            for b in range(NBUF):
                store_wait(g + b, b)
                gather_start(g + b + NBUF, b)

        g_last = rows_per_w - NBUF
        for b in range(NBUF):
            gather_wait(g_last + b, b)
            store_start(g_last + b, b)
        for b in range(NBUF):
            store_wait(g_last + b, b)

    return k


def kernel(x, table):
    b, l = x.shape
    flat = x.reshape(-1)
    n_rows = flat.shape[0] // IDXW
    idx2d = flat.reshape(n_rows, IDXW)
    out = _make_kernel(n_rows)(idx2d, table)
    return out.reshape(b, l, DIM)
